# Initial kernel scaffold; baseline (speedup 1.0000x reference)
import jax
import jax.numpy as jnp
from jax.experimental import pallas as pl


def kernel(boxes, scores):
    raise NotImplementedError("write your pallas kernel here")


# speedup vs baseline: 375.3862x; 375.3862x over previous
import jax
import jax.numpy as jnp
from jax import lax
from jax.experimental import pallas as pl
from jax.experimental.pallas import tpu as pltpu

N_IN = 20000
NMS_PRE = 4096
NMS_POST = 500
SCORE_THR = 0.1
NMS_THR = 0.1

C = 256
NCHUNK = NMS_PRE // C
KBUF = 1024

_HI = jax.lax.Precision.HIGHEST


def _nms_body(cand_ref, candt_ref, out_ref, kept_ref, alive_ref, alive0_ref,
              adj_ref):
    kept_ref[...] = jnp.zeros((8, KBUF), jnp.float32)

    def chunk_body(carry):
        t, count = carry

        cand = cand_ref[t]
        candt = candt_ref[t]

        cx = cand[:, 0:1]
        cy = cand[:, 1:2]
        cdx = cand[:, 3:4]
        cdy = cand[:, 4:5]
        cs = cand[:, 7:8]
        cx1 = cx - cdx * 0.5
        cx2 = cx + cdx * 0.5
        cy1 = cy - cdy * 0.5
        cy2 = cy + cdy * 0.5
        carea = cdx * cdy

        kx = kept_ref[0:1, :]
        ky = kept_ref[1:2, :]
        kdx = kept_ref[3:4, :]
        kdy = kept_ref[4:5, :]
        kx1 = kx - kdx * 0.5
        kx2 = kx + kdx * 0.5
        ky1 = ky - kdy * 0.5
        ky2 = ky + kdy * 0.5
        karea = kdx * kdy

        ix = jnp.maximum(0.0, jnp.minimum(cx2, kx2) - jnp.maximum(cx1, kx1))
        iy = jnp.maximum(0.0, jnp.minimum(cy2, ky2) - jnp.maximum(cy1, ky1))
        inter = ix * iy
        union = carea + karea - inter
        iou = inter / jnp.maximum(union, 1e-6)
        supp = jnp.any(iou > NMS_THR, axis=1, keepdims=True)

        alive0 = jnp.where((cs > SCORE_THR) & ~supp, 1.0, 0.0)

        rx = candt[0:1, :]
        ry = candt[1:2, :]
        rdx = candt[3:4, :]
        rdy = candt[4:5, :]
        rx1 = rx - rdx * 0.5
        rx2 = rx + rdx * 0.5
        ry1 = ry - rdy * 0.5
        ry2 = ry + rdy * 0.5
        rarea = rdx * rdy

        ixc = jnp.maximum(0.0, jnp.minimum(cx2, rx2) - jnp.maximum(cx1, rx1))
        iyc = jnp.maximum(0.0, jnp.minimum(cy2, ry2) - jnp.maximum(cy1, ry1))
        interc = ixc * iyc
        unionc = carea + rarea - interc
        iouc = interc / jnp.maximum(unionc, 1e-6)
        row_i = lax.broadcasted_iota(jnp.int32, (C, C), 0)
        col_i = lax.broadcasted_iota(jnp.int32, (C, C), 1)
        adj_ref[...] = jnp.where((iouc > NMS_THR) & (row_i > col_i), 1.0, 0.0)
        alive0_ref[...] = alive0
        alive_ref[...] = alive0

        def inner_body(_):
            alive = alive_ref[...]
            sup = jax.lax.dot_general(
                adj_ref[...], alive,
                (((1,), (0,)), ((), ())), precision=_HI)
            new = jnp.where(sup > 0.5, 0.0, alive0_ref[...])
            alive_ref[...] = new
            return (jnp.sum(jnp.abs(new - alive)) > 0).astype(jnp.int32)

        lax.while_loop(lambda ch: ch > 0, inner_body, jnp.int32(1))
        alive = alive_ref[...]

        lower = jnp.where(row_i > col_i, 1.0, 0.0)
        pos = count.astype(jnp.float32) + jax.lax.dot_general(
            lower, alive, (((1,), (0,)), ((), ())), precision=_HI)
        slot = lax.broadcasted_iota(jnp.int32, (C, KBUF), 1).astype(jnp.float32)
        onehot = jnp.where((slot == pos) & (alive > 0.5), 1.0, 0.0)
        app = jax.lax.dot_general(
            candt, onehot, (((1,), (0,)), ((), ())), precision=_HI)
        kept_ref[...] = kept_ref[...] + app

        na = jnp.sum(alive).astype(jnp.int32)
        return t + 1, count + na

    lax.while_loop(
        lambda carry: (carry[0] < NCHUNK) & (carry[1] < NMS_POST),
        chunk_body, (jnp.int32(0), jnp.int32(0)))

    out_ref[...] = kept_ref[:, 0:512]


def _run_nms(cand, candt):
    return pl.pallas_call(
        _nms_body,
        out_shape=jax.ShapeDtypeStruct((8, 512), jnp.float32),
        scratch_shapes=[
            pltpu.VMEM((8, KBUF), jnp.float32),
            pltpu.VMEM((C, 1), jnp.float32),
            pltpu.VMEM((C, 1), jnp.float32),
            pltpu.VMEM((C, C), jnp.float32),
        ],
    )(cand, candt)


def kernel(boxes, scores):
    masked = jnp.where(scores > SCORE_THR, scores, -jnp.inf)
    top_scores, top_idx = jax.lax.top_k(masked, NMS_PRE)
    sel = jnp.take(boxes, top_idx, axis=0)
    ts_clean = jnp.where(jnp.isfinite(top_scores), top_scores, 0.0)
    cand8 = jnp.concatenate([sel, ts_clean[:, None]], axis=-1)
    cand = cand8.reshape(NCHUNK, C, 8)
    candt = jnp.transpose(cand, (0, 2, 1))
    outt = _run_nms(cand, candt)
    return outt[:, :NMS_POST].T
